# Initial kernel scaffold; baseline (speedup 1.0000x reference)
import jax
import jax.numpy as jnp
from jax.experimental import pallas as pl


def kernel(h, x, t, edge_index, W_e1, b_e1, W_e2, b_e2, W_n1, b_n1, W_n2, b_n2, W_c1, b_c1, W_c2):
    raise NotImplementedError("write your pallas kernel here")


# speedup vs baseline: 1.8412x; 1.8412x over previous
import functools

import jax
import jax.numpy as jnp
from jax import lax
from jax.experimental import pallas as pl
from jax.experimental.pallas import tpu as pltpu
from jax.experimental.pallas import tpu_sc as plsc

N = 10000
E = 320000
NF = 128
L = 4
XW = 16
W = NF + XW

NC = 2
NS = 16
NWORK = NC * NS
EPW = E // NWORK
K = 80
NCHUNK = EPW // K
RPT = N // NS

BE = 2000
BN = 2000

_sc_mesh = plsc.VectorSubcoreMesh(core_axis_name="c", subcore_axis_name="s")


def _silu(v):
    return v * (1.0 / (1.0 + jnp.exp(-v)))


@functools.partial(
    pl.kernel,
    mesh=_sc_mesh,
    out_type=[
        jax.ShapeDtypeStruct((E, W), jnp.float32),
        jax.ShapeDtypeStruct((E, W), jnp.float32),
    ],
    scratch_types=[
        pltpu.VMEM((K,), jnp.int32),
        pltpu.VMEM((K,), jnp.int32),
        pltpu.VMEM((K, W), jnp.float32),
        pltpu.VMEM((K, W), jnp.float32),
        pltpu.SemaphoreType.DMA,
        pltpu.SemaphoreType.DMA,
    ],
    compiler_params=pltpu.CompilerParams(use_tc_tiling_on_sc=False),
)
def _sc_gather(row_hbm, col_hbm, thr_hbm, thc_hbm, outr_hbm, outc_hbm,
               idxr_v, idxc_v, bufr_v, bufc_v, semr, semc):
    wid = lax.axis_index("s") * NC + lax.axis_index("c")
    base = wid * EPW

    def body(i, carry):
        off = base + i * K
        pltpu.sync_copy(row_hbm.at[pl.ds(off, K)], idxr_v)
        pltpu.sync_copy(col_hbm.at[pl.ds(off, K)], idxc_v)
        cr = pltpu.async_copy(thr_hbm.at[idxr_v], bufr_v, semr)
        cc = pltpu.async_copy(thc_hbm.at[idxc_v], bufc_v, semc)
        cr.wait()
        cc.wait()
        pltpu.sync_copy(bufr_v, outr_hbm.at[pl.ds(off, K)])
        pltpu.sync_copy(bufc_v, outc_hbm.at[pl.ds(off, K)])
        return carry

    lax.fori_loop(0, NCHUNK, body, 0)


@functools.partial(
    pl.kernel,
    mesh=_sc_mesh,
    out_type=jax.ShapeDtypeStruct((2 * N, W), jnp.float32),
    scratch_types=[
        pltpu.VMEM((K,), jnp.int32),
        pltpu.VMEM((K, W), jnp.float32),
        pltpu.VMEM_SHARED((N, W), jnp.float32),
    ],
    compiler_params=pltpu.CompilerParams(use_tc_tiling_on_sc=False),
)
def _sc_scatter(row_hbm, y_hbm, zero_hbm, out_hbm, idx_v, ybuf_v, acc_sh):
    cid = lax.axis_index("c")
    sid = lax.axis_index("s")
    wid = sid * NC + cid
    pltpu.sync_copy(zero_hbm.at[pl.ds(sid * RPT, RPT)],
                    acc_sh.at[pl.ds(sid * RPT, RPT)])
    plsc.subcore_barrier()
    base = wid * EPW

    def body(i, carry):
        off = base + i * K
        pltpu.sync_copy(row_hbm.at[pl.ds(off, K)], idx_v)
        pltpu.sync_copy(y_hbm.at[pl.ds(off, K)], ybuf_v)
        pltpu.sync_copy(ybuf_v, acc_sh.at[idx_v], add=True)
        return carry

    lax.fori_loop(0, NCHUNK, body, 0)
    plsc.subcore_barrier()
    pltpu.sync_copy(acc_sh.at[pl.ds(sid * RPT, RPT)],
                    out_hbm.at[pl.ds(cid * N + sid * RPT, RPT)])


def _prep_body(wea_ref, web_ref, h_ref, xp_ref, thr_ref, thc_ref):
    h = h_ref[...]
    xp = xp_ref[...]
    thr_ref[...] = jnp.concatenate(
        [jnp.dot(h, wea_ref[...], preferred_element_type=jnp.float32), xp], axis=1)
    thc_ref[...] = jnp.concatenate(
        [jnp.dot(h, web_ref[...], preferred_element_type=jnp.float32), xp], axis=1)


def _edge_body(wd_ref, cvec_ref, we2_ref, be2_ref, wc1_ref, bc1_ref, wc2_ref,
               r_ref, c_ref, out_ref):
    r = r_ref[...]
    c = c_ref[...]
    pre = r[:, :NF] + c[:, :NF]
    diff = r[:, NF:] - c[:, NF:]
    d2 = jnp.sum(diff * diff, axis=1, keepdims=True)
    u = pre + d2 * wd_ref[...] + cvec_ref[...]
    m1 = _silu(u)
    m = _silu(jnp.dot(m1, we2_ref[...], preferred_element_type=jnp.float32)
              + be2_ref[...])
    cm = _silu(jnp.dot(m, wc1_ref[...], preferred_element_type=jnp.float32)
               + bc1_ref[...])
    cc = jnp.sum(cm * wc2_ref[...], axis=1, keepdims=True)
    scale = cc / (jnp.sqrt(d2 + 1e-8) + 1.0)
    out_ref[...] = jnp.concatenate([m, diff * scale], axis=1)


def _node_body(wn1a_ref, wn1b_ref, bn1_ref, wn2_ref, bn2_ref, wea_ref, web_ref,
               h_ref, xp_ref, p0_ref, p1_ref,
               h_out, xp_out, thr_ref, thc_ref):
    h = h_ref[...]
    agg = p0_ref[:, :NF] + p1_ref[:, :NF]
    n1 = _silu(jnp.dot(h, wn1a_ref[...], preferred_element_type=jnp.float32)
               + jnp.dot(agg, wn1b_ref[...], preferred_element_type=jnp.float32)
               + bn1_ref[...])
    hn = h + jnp.dot(n1, wn2_ref[...], preferred_element_type=jnp.float32) \
        + bn2_ref[...]
    xn = xp_ref[...] + (p0_ref[:, NF:] + p1_ref[:, NF:])
    h_out[...] = hn
    xp_out[...] = xn
    thr_ref[...] = jnp.concatenate(
        [jnp.dot(hn, wea_ref[...], preferred_element_type=jnp.float32), xn], axis=1)
    thc_ref[...] = jnp.concatenate(
        [jnp.dot(hn, web_ref[...], preferred_element_type=jnp.float32), xn], axis=1)


def _center_body(xp_ref, out_ref):
    xp = xp_ref[...]
    out_ref[...] = xp - jnp.mean(xp, axis=0, keepdims=True)


def _full(shape):
    return pl.BlockSpec(shape, lambda i: (0,) * len(shape))


def _rows(shape):
    return pl.BlockSpec(shape, lambda i: (i,) + (0,) * (len(shape) - 1))


def _prep_call(wea, web, h, xp):
    return pl.pallas_call(
        _prep_body,
        grid=(N // BN,),
        in_specs=[_full((NF, NF)), _full((NF, NF)), _rows((BN, NF)), _rows((BN, XW))],
        out_specs=[_rows((BN, W)), _rows((BN, W))],
        out_shape=[jax.ShapeDtypeStruct((N, W), jnp.float32)] * 2,
    )(wea, web, h, xp)


def _edge_call(wd, cvec, we2, be2, wc1, bc1, wc2, r, c):
    return pl.pallas_call(
        _edge_body,
        grid=(E // BE,),
        in_specs=[_full((1, NF)), _full((1, NF)), _full((NF, NF)), _full((1, NF)),
                  _full((NF, NF)), _full((1, NF)), _full((1, NF)),
                  _rows((BE, W)), _rows((BE, W))],
        out_specs=_rows((BE, W)),
        out_shape=jax.ShapeDtypeStruct((E, W), jnp.float32),
    )(wd, cvec, we2, be2, wc1, bc1, wc2, r, c)


def _node_call(wn1a, wn1b, bn1, wn2, bn2, wea, web, h, xp, p0, p1):
    return pl.pallas_call(
        _node_body,
        grid=(N // BN,),
        in_specs=[_full((NF, NF)), _full((NF, NF)), _full((1, NF)),
                  _full((NF, NF)), _full((1, NF)), _full((NF, NF)), _full((NF, NF)),
                  _rows((BN, NF)), _rows((BN, XW)), _rows((BN, W)), _rows((BN, W))],
        out_specs=[_rows((BN, NF)), _rows((BN, XW)), _rows((BN, W)), _rows((BN, W))],
        out_shape=[jax.ShapeDtypeStruct((N, NF), jnp.float32),
                   jax.ShapeDtypeStruct((N, XW), jnp.float32),
                   jax.ShapeDtypeStruct((N, W), jnp.float32),
                   jax.ShapeDtypeStruct((N, W), jnp.float32)],
    )(wn1a, wn1b, bn1, wn2, bn2, wea, web, h, xp, p0, p1)


def _center_call(xp):
    return pl.pallas_call(
        _center_body,
        out_shape=jax.ShapeDtypeStruct((N, XW), jnp.float32),
    )(xp)


def kernel(h, x, t, edge_index, W_e1, b_e1, W_e2, b_e2, W_n1, b_n1, W_n2, b_n2,
           W_c1, b_c1, W_c2):
    row = edge_index[0]
    col = edge_index[1]
    xp = jnp.pad(x, ((0, 0), (0, XW - 3)))
    zeros_nw = jnp.zeros((N, W), jnp.float32)

    thr, thc = _prep_call(W_e1[0][:NF], W_e1[0][NF:2 * NF], h, xp)
    for l in range(L):
        wd = W_e1[l][2 * NF][None, :]
        cvec = (t[0] * W_e1[l][2 * NF + 1] + b_e1[l])[None, :]
        r, c = _sc_gather(row, col, thr, thc)
        y = _edge_call(wd, cvec, W_e2[l], b_e2[l][None, :], W_c1[l],
                       b_c1[l][None, :], W_c2[l][:, 0][None, :], r, c)
        p = _sc_scatter(row, y, zeros_nw)
        la = min(l + 1, L - 1)
        h, xp, thr, thc = _node_call(
            W_n1[l][:NF], W_n1[l][NF:], b_n1[l][None, :], W_n2[l],
            b_n2[l][None, :], W_e1[la][:NF], W_e1[la][NF:2 * NF],
            h, xp, p[:N], p[N:])
    xo = _center_call(xp)
    return xo[:, :3]


# speedup vs baseline: 2.6177x; 1.4217x over previous
import functools

import jax
import jax.numpy as jnp
from jax import lax
from jax.experimental import pallas as pl
from jax.experimental.pallas import tpu as pltpu
from jax.experimental.pallas import tpu_sc as plsc

N = 10000
E = 320000
NF = 128
L = 4
XW = 16
W = NF + XW

NC = 2
NS = 16
NWORK = NC * NS
EPW = E // NWORK
K = 80
NCHUNK = EPW // K
NPAIR = NCHUNK // 2
RPT = N // NS

BE = 2000
BN = 2000

_sc_mesh = plsc.VectorSubcoreMesh(core_axis_name="c", subcore_axis_name="s")
_sc_params = pltpu.CompilerParams(use_tc_tiling_on_sc=False)


def _silu(v):
    return v * (0.5 * jnp.tanh(0.5 * v) + 0.5)


@functools.partial(
    pl.kernel,
    mesh=_sc_mesh,
    out_type=[
        jax.ShapeDtypeStruct((E, W), jnp.float32),
        jax.ShapeDtypeStruct((E, W), jnp.float32),
    ],
    scratch_types=[
        pltpu.VMEM((NCHUNK, K), jnp.int32),
        pltpu.VMEM((NCHUNK, K), jnp.int32),
        pltpu.VMEM((K, W), jnp.float32),
        pltpu.VMEM((K, W), jnp.float32),
        pltpu.VMEM((K, W), jnp.float32),
        pltpu.VMEM((K, W), jnp.float32),
        pltpu.SemaphoreType.DMA,
        pltpu.SemaphoreType.DMA,
        pltpu.SemaphoreType.DMA,
        pltpu.SemaphoreType.DMA,
        pltpu.SemaphoreType.DMA,
        pltpu.SemaphoreType.DMA,
    ],
    compiler_params=_sc_params,
)
def _sc_gather(row_hbm, col_hbm, thr_hbm, thc_hbm, outr_hbm, outc_hbm,
               idxr_v, idxc_v, bufr0, bufc0, bufr1, bufc1,
               semr0, semc0, semr1, semc1, semw0, semw1):
    wid = lax.axis_index("s") * NC + lax.axis_index("c")
    base = wid * EPW
    pltpu.sync_copy(row_hbm.at[wid], idxr_v)
    pltpu.sync_copy(col_hbm.at[wid], idxc_v)

    def gather(ch, bufr, bufc, semr, semc):
        cr = pltpu.async_copy(thr_hbm.at[idxr_v.at[ch]], bufr, semr)
        cc = pltpu.async_copy(thc_hbm.at[idxc_v.at[ch]], bufc, semc)
        return cr, cc

    def wait_gather(ch, bufr, bufc, semr, semc):
        pltpu.make_async_copy(thr_hbm.at[idxr_v.at[ch]], bufr, semr).wait()
        pltpu.make_async_copy(thc_hbm.at[idxc_v.at[ch]], bufc, semc).wait()

    def write(ch, bufr, bufc, semw):
        off = base + ch * K
        pltpu.async_copy(bufr, outr_hbm.at[pl.ds(off, K)], semw)
        pltpu.async_copy(bufc, outc_hbm.at[pl.ds(off, K)], semw)

    def wait_write(ch, bufr, bufc, semw):
        off = base + ch * K
        pltpu.make_async_copy(bufr, outr_hbm.at[pl.ds(off, K)], semw).wait()
        pltpu.make_async_copy(bufc, outc_hbm.at[pl.ds(off, K)], semw).wait()

    gather(0, bufr0, bufc0, semr0, semc0)

    def body(i, carry):
        c0 = 2 * i
        @pl.when(i > 0)
        def _():
            wait_write(c0 - 1, bufr1, bufc1, semw1)
        gather(c0 + 1, bufr1, bufc1, semr1, semc1)
        wait_gather(c0, bufr0, bufc0, semr0, semc0)
        write(c0, bufr0, bufc0, semw0)
        wait_write(c0, bufr0, bufc0, semw0)
        @pl.when(c0 + 2 < NCHUNK)
        def _():
            gather(c0 + 2, bufr0, bufc0, semr0, semc0)
        wait_gather(c0 + 1, bufr1, bufc1, semr1, semc1)
        write(c0 + 1, bufr1, bufc1, semw1)
        return carry

    lax.fori_loop(0, NPAIR, body, 0)
    wait_write(NCHUNK - 2, bufr1, bufc1, semw1)
    wait_gather(NCHUNK - 1, bufr0, bufc0, semr0, semc0)
    write(NCHUNK - 1, bufr0, bufc0, semw0)
    wait_write(NCHUNK - 1, bufr0, bufc0, semw0)


@functools.partial(
    pl.kernel,
    mesh=_sc_mesh,
    out_type=[
        jax.ShapeDtypeStruct((2 * N, NF), jnp.float32),
        jax.ShapeDtypeStruct((2 * N, XW), jnp.float32),
    ],
    scratch_types=[
        pltpu.VMEM((NCHUNK, K), jnp.int32),
        pltpu.VMEM((K, NF), jnp.float32),
        pltpu.VMEM((K, NF), jnp.float32),
        pltpu.VMEM((K, XW), jnp.float32),
        pltpu.VMEM((K, XW), jnp.float32),
        pltpu.VMEM_SHARED((N, NF), jnp.float32),
        pltpu.VMEM_SHARED((N, XW), jnp.float32),
        pltpu.SemaphoreType.DMA,
        pltpu.SemaphoreType.DMA,
        pltpu.SemaphoreType.DMA,
        pltpu.SemaphoreType.DMA,
    ],
    compiler_params=_sc_params,
)
def _sc_scatter(row_hbm, m_hbm, t_hbm, zm_hbm, zt_hbm, outm_hbm, outt_hbm,
                idx_v, mbuf0, mbuf1, tbuf0, tbuf1, accm, acct,
                semy0, semy1, sems0, sems1):
    cid = lax.axis_index("c")
    sid = lax.axis_index("s")
    wid = sid * NC + cid
    pltpu.sync_copy(zm_hbm.at[pl.ds(sid * RPT, RPT)],
                    accm.at[pl.ds(sid * RPT, RPT)])
    pltpu.sync_copy(zt_hbm.at[pl.ds(sid * RPT, RPT)],
                    acct.at[pl.ds(sid * RPT, RPT)])
    pltpu.sync_copy(row_hbm.at[wid], idx_v)
    plsc.subcore_barrier()
    base = wid * EPW

    def read(ch, mbuf, tbuf, semy):
        off = base + ch * K
        pltpu.async_copy(m_hbm.at[pl.ds(off, K)], mbuf, semy)
        pltpu.async_copy(t_hbm.at[pl.ds(off, K)], tbuf, semy)

    def wait_read(ch, mbuf, tbuf, semy):
        off = base + ch * K
        pltpu.make_async_copy(m_hbm.at[pl.ds(off, K)], mbuf, semy).wait()
        pltpu.make_async_copy(t_hbm.at[pl.ds(off, K)], tbuf, semy).wait()

    def scat(ch, mbuf, tbuf, sems):
        pltpu.async_copy(mbuf, accm.at[idx_v.at[ch]], sems, add=True)
        pltpu.async_copy(tbuf, acct.at[idx_v.at[ch]], sems, add=True)

    def wait_scat(ch, mbuf, tbuf, sems):
        pltpu.make_async_copy(mbuf, accm.at[idx_v.at[ch]], sems).wait()
        pltpu.make_async_copy(tbuf, acct.at[idx_v.at[ch]], sems).wait()

    read(0, mbuf0, tbuf0, semy0)

    def body(i, carry):
        c0 = 2 * i
        @pl.when(i > 0)
        def _():
            wait_scat(c0 - 1, mbuf1, tbuf1, sems1)
        read(c0 + 1, mbuf1, tbuf1, semy1)
        wait_read(c0, mbuf0, tbuf0, semy0)
        scat(c0, mbuf0, tbuf0, sems0)
        wait_scat(c0, mbuf0, tbuf0, sems0)
        @pl.when(c0 + 2 < NCHUNK)
        def _():
            read(c0 + 2, mbuf0, tbuf0, semy0)
        wait_read(c0 + 1, mbuf1, tbuf1, semy1)
        scat(c0 + 1, mbuf1, tbuf1, sems1)
        return carry

    lax.fori_loop(0, NPAIR, body, 0)
    wait_scat(NCHUNK - 2, mbuf1, tbuf1, sems1)
    wait_read(NCHUNK - 1, mbuf0, tbuf0, semy0)
    scat(NCHUNK - 1, mbuf0, tbuf0, sems0)
    wait_scat(NCHUNK - 1, mbuf0, tbuf0, sems0)
    plsc.subcore_barrier()
    pltpu.sync_copy(accm.at[pl.ds(sid * RPT, RPT)],
                    outm_hbm.at[pl.ds(cid * N + sid * RPT, RPT)])
    pltpu.sync_copy(acct.at[pl.ds(sid * RPT, RPT)],
                    outt_hbm.at[pl.ds(cid * N + sid * RPT, RPT)])


@functools.partial(
    pl.kernel,
    mesh=_sc_mesh,
    out_type=jax.ShapeDtypeStruct((2 * N, XW), jnp.float32),
    scratch_types=[
        pltpu.VMEM((NCHUNK, K), jnp.int32),
        pltpu.VMEM((K, XW), jnp.float32),
        pltpu.VMEM((K, XW), jnp.float32),
        pltpu.VMEM_SHARED((N, XW), jnp.float32),
        pltpu.SemaphoreType.DMA,
        pltpu.SemaphoreType.DMA,
        pltpu.SemaphoreType.DMA,
        pltpu.SemaphoreType.DMA,
    ],
    compiler_params=_sc_params,
)
def _sc_scatter_t(row_hbm, t_hbm, zt_hbm, outt_hbm,
                  idx_v, tbuf0, tbuf1, acct, semy0, semy1, sems0, sems1):
    cid = lax.axis_index("c")
    sid = lax.axis_index("s")
    wid = sid * NC + cid
    pltpu.sync_copy(zt_hbm.at[pl.ds(sid * RPT, RPT)],
                    acct.at[pl.ds(sid * RPT, RPT)])
    pltpu.sync_copy(row_hbm.at[wid], idx_v)
    plsc.subcore_barrier()
    base = wid * EPW

    def read(ch, tbuf, semy):
        pltpu.async_copy(t_hbm.at[pl.ds(base + ch * K, K)], tbuf, semy)

    def wait_read(ch, tbuf, semy):
        pltpu.make_async_copy(t_hbm.at[pl.ds(base + ch * K, K)], tbuf, semy).wait()

    def scat(ch, tbuf, sems):
        pltpu.async_copy(tbuf, acct.at[idx_v.at[ch]], sems, add=True)

    def wait_scat(ch, tbuf, sems):
        pltpu.make_async_copy(tbuf, acct.at[idx_v.at[ch]], sems).wait()

    read(0, tbuf0, semy0)

    def body(i, carry):
        c0 = 2 * i
        @pl.when(i > 0)
        def _():
            wait_scat(c0 - 1, tbuf1, sems1)
        read(c0 + 1, tbuf1, semy1)
        wait_read(c0, tbuf0, semy0)
        scat(c0, tbuf0, sems0)
        wait_scat(c0, tbuf0, sems0)
        @pl.when(c0 + 2 < NCHUNK)
        def _():
            read(c0 + 2, tbuf0, semy0)
        wait_read(c0 + 1, tbuf1, semy1)
        scat(c0 + 1, tbuf1, sems1)
        return carry

    lax.fori_loop(0, NPAIR, body, 0)
    wait_scat(NCHUNK - 2, tbuf1, sems1)
    wait_read(NCHUNK - 1, tbuf0, semy0)
    scat(NCHUNK - 1, tbuf0, sems0)
    wait_scat(NCHUNK - 1, tbuf0, sems0)
    plsc.subcore_barrier()
    pltpu.sync_copy(acct.at[pl.ds(sid * RPT, RPT)],
                    outt_hbm.at[pl.ds(cid * N + sid * RPT, RPT)])


def _prep_body(wea_ref, web_ref, h_ref, xp_ref, thr_ref, thc_ref):
    h = h_ref[...]
    xp = xp_ref[...]
    thr_ref[...] = jnp.concatenate(
        [jnp.dot(h, wea_ref[...], preferred_element_type=jnp.float32), xp], axis=1)
    thc_ref[...] = jnp.concatenate(
        [jnp.dot(h, web_ref[...], preferred_element_type=jnp.float32), xp], axis=1)


def _edge_common(wd_ref, cvec_ref, we2_ref, be2_ref, wc1_ref, bc1_ref, wc2_ref,
                 r_ref, c_ref):
    r = r_ref[...]
    c = c_ref[...]
    pre = r[:, :NF] + c[:, :NF]
    diff = r[:, NF:] - c[:, NF:]
    d2 = jnp.sum(diff * diff, axis=1, keepdims=True)
    u = pre + d2 * wd_ref[...] + cvec_ref[...]
    m1 = _silu(u)
    m = _silu(jnp.dot(m1, we2_ref[...], preferred_element_type=jnp.float32)
              + be2_ref[...])
    cm = _silu(jnp.dot(m, wc1_ref[...], preferred_element_type=jnp.float32)
               + bc1_ref[...])
    cc = jnp.sum(cm * wc2_ref[...], axis=1, keepdims=True)
    s = d2 + 1e-8
    norm = s * lax.rsqrt(s)
    trans = diff * (cc / (norm + 1.0))
    return m, trans


def _edge_body(wd_ref, cvec_ref, we2_ref, be2_ref, wc1_ref, bc1_ref, wc2_ref,
               r_ref, c_ref, m_ref, t_ref):
    m, trans = _edge_common(wd_ref, cvec_ref, we2_ref, be2_ref, wc1_ref,
                            bc1_ref, wc2_ref, r_ref, c_ref)
    m_ref[...] = m
    t_ref[...] = trans


def _edge_body_t(wd_ref, cvec_ref, we2_ref, be2_ref, wc1_ref, bc1_ref, wc2_ref,
                 r_ref, c_ref, t_ref):
    _, trans = _edge_common(wd_ref, cvec_ref, we2_ref, be2_ref, wc1_ref,
                            bc1_ref, wc2_ref, r_ref, c_ref)
    t_ref[...] = trans


def _node_body(wn1a_ref, wn1b_ref, bn1_ref, wn2_ref, bn2_ref, wea_ref, web_ref,
               h_ref, xp_ref, p0m_ref, p1m_ref, p0t_ref, p1t_ref,
               h_out, xp_out, thr_ref, thc_ref):
    h = h_ref[...]
    agg = p0m_ref[...] + p1m_ref[...]
    n1 = _silu(jnp.dot(h, wn1a_ref[...], preferred_element_type=jnp.float32)
               + jnp.dot(agg, wn1b_ref[...], preferred_element_type=jnp.float32)
               + bn1_ref[...])
    hn = h + jnp.dot(n1, wn2_ref[...], preferred_element_type=jnp.float32) \
        + bn2_ref[...]
    xn = xp_ref[...] + (p0t_ref[...] + p1t_ref[...])
    h_out[...] = hn
    xp_out[...] = xn
    thr_ref[...] = jnp.concatenate(
        [jnp.dot(hn, wea_ref[...], preferred_element_type=jnp.float32), xn], axis=1)
    thc_ref[...] = jnp.concatenate(
        [jnp.dot(hn, web_ref[...], preferred_element_type=jnp.float32), xn], axis=1)


def _final_body(xp_ref, p0t_ref, p1t_ref, out_ref):
    xn = xp_ref[...] + p0t_ref[...] + p1t_ref[...]
    out_ref[...] = xn - jnp.mean(xn, axis=0, keepdims=True)


def _full(shape):
    return pl.BlockSpec(shape, lambda i: (0,) * len(shape))


def _rows(shape):
    return pl.BlockSpec(shape, lambda i: (i,) + (0,) * (len(shape) - 1))


def _prep_call(wea, web, h, xp):
    return pl.pallas_call(
        _prep_body,
        grid=(N // BN,),
        in_specs=[_full((NF, NF)), _full((NF, NF)), _rows((BN, NF)), _rows((BN, XW))],
        out_specs=[_rows((BN, W)), _rows((BN, W))],
        out_shape=[jax.ShapeDtypeStruct((N, W), jnp.float32)] * 2,
    )(wea, web, h, xp)


_EDGE_WSPECS = [_full((1, NF)), _full((1, NF)), _full((NF, NF)), _full((1, NF)),
                _full((NF, NF)), _full((1, NF)), _full((1, NF))]


def _edge_call(wd, cvec, we2, be2, wc1, bc1, wc2, r, c):
    return pl.pallas_call(
        _edge_body,
        grid=(E // BE,),
        in_specs=_EDGE_WSPECS + [_rows((BE, W)), _rows((BE, W))],
        out_specs=[_rows((BE, NF)), _rows((BE, XW))],
        out_shape=[jax.ShapeDtypeStruct((E, NF), jnp.float32),
                   jax.ShapeDtypeStruct((E, XW), jnp.float32)],
    )(wd, cvec, we2, be2, wc1, bc1, wc2, r, c)


def _edge_call_t(wd, cvec, we2, be2, wc1, bc1, wc2, r, c):
    return pl.pallas_call(
        _edge_body_t,
        grid=(E // BE,),
        in_specs=_EDGE_WSPECS + [_rows((BE, W)), _rows((BE, W))],
        out_specs=_rows((BE, XW)),
        out_shape=jax.ShapeDtypeStruct((E, XW), jnp.float32),
    )(wd, cvec, we2, be2, wc1, bc1, wc2, r, c)


def _node_call(wn1a, wn1b, bn1, wn2, bn2, wea, web, h, xp, pm, pt):
    return pl.pallas_call(
        _node_body,
        grid=(N // BN,),
        in_specs=[_full((NF, NF)), _full((NF, NF)), _full((1, NF)),
                  _full((NF, NF)), _full((1, NF)), _full((NF, NF)), _full((NF, NF)),
                  _rows((BN, NF)), _rows((BN, XW)),
                  pl.BlockSpec((BN, NF), lambda i: (i, 0)),
                  pl.BlockSpec((BN, NF), lambda i: (i + N // BN, 0)),
                  pl.BlockSpec((BN, XW), lambda i: (i, 0)),
                  pl.BlockSpec((BN, XW), lambda i: (i + N // BN, 0))],
        out_specs=[_rows((BN, NF)), _rows((BN, XW)), _rows((BN, W)), _rows((BN, W))],
        out_shape=[jax.ShapeDtypeStruct((N, NF), jnp.float32),
                   jax.ShapeDtypeStruct((N, XW), jnp.float32),
                   jax.ShapeDtypeStruct((N, W), jnp.float32),
                   jax.ShapeDtypeStruct((N, W), jnp.float32)],
    )(wn1a, wn1b, bn1, wn2, bn2, wea, web, h, xp, pm, pm, pt, pt)


def _final_call(xp, pt):
    return pl.pallas_call(
        _final_body,
        grid=(1,),
        in_specs=[pl.BlockSpec((N, XW), lambda i: (0, 0)),
                  pl.BlockSpec((N, XW), lambda i: (0, 0)),
                  pl.BlockSpec((N, XW), lambda i: (1, 0))],
        out_specs=pl.BlockSpec((N, XW), lambda i: (0, 0)),
        out_shape=jax.ShapeDtypeStruct((N, XW), jnp.float32),
    )(xp, pt, pt)


def kernel(h, x, t, edge_index, W_e1, b_e1, W_e2, b_e2, W_n1, b_n1, W_n2, b_n2,
           W_c1, b_c1, W_c2):
    row = edge_index[0]
    col = edge_index[1]
    row3 = row.reshape(NWORK, NCHUNK, K)
    col3 = col.reshape(NWORK, NCHUNK, K)
    xp = jnp.pad(x, ((0, 0), (0, XW - 3)))
    zm = jnp.zeros((N, NF), jnp.float32)
    zt = jnp.zeros((N, XW), jnp.float32)

    thr, thc = _prep_call(W_e1[0][:NF], W_e1[0][NF:2 * NF], h, xp)
    for l in range(L):
        wd = W_e1[l][2 * NF][None, :]
        cvec = (t[0] * W_e1[l][2 * NF + 1] + b_e1[l])[None, :]
        r, c = _sc_gather(row3, col3, thr, thc)
        eargs = (wd, cvec, W_e2[l], b_e2[l][None, :], W_c1[l],
                 b_c1[l][None, :], W_c2[l][:, 0][None, :], r, c)
        if l < L - 1:
            m, tr = _edge_call(*eargs)
            pm, pt = _sc_scatter(row3, m, tr, zm, zt)
            h, xp, thr, thc = _node_call(
                W_n1[l][:NF], W_n1[l][NF:], b_n1[l][None, :], W_n2[l],
                b_n2[l][None, :], W_e1[l + 1][:NF], W_e1[l + 1][NF:2 * NF],
                h, xp, pm, pt)
        else:
            tr = _edge_call_t(*eargs)
            pt = _sc_scatter_t(row3, tr, zt)
            xo = _final_call(xp, pt)
    return xo[:, :3]
